# Initial kernel scaffold; baseline (speedup 1.0000x reference)
import jax
import jax.numpy as jnp
from jax.experimental import pallas as pl


def kernel(x):
    raise NotImplementedError("write your pallas kernel here")


# speedup vs baseline: 29.1388x; 29.1388x over previous
import jax
import jax.numpy as jnp
from jax import lax
from jax.experimental import pallas as pl
from jax.experimental.pallas import tpu as pltpu

_NB = 256
_R = 16
_CB = 4096
_HP = lax.Precision.HIGHEST


def _hist_kernel(x_ref, gray_ref, acc_ref):
    c = pl.program_id(1)
    xs = x_ref[0]
    u8r = jnp.clip(jnp.floor(xs[0] * 255.0), 0.0, 255.0)
    u8g = jnp.clip(jnp.floor(xs[1] * 255.0), 0.0, 255.0)
    u8b = jnp.clip(jnp.floor(xs[2] * 255.0), 0.0, 255.0)
    grayf = jnp.clip(jnp.round(0.299 * u8r + 0.587 * u8g + 0.114 * u8b),
                     0.0, 255.0)
    gray_ref[0] = grayf.astype(jnp.bfloat16)

    hi = jnp.floor(grayf * 0.0625)
    lo = grayf - hi * 16.0
    hrep = pltpu.repeat(hi.astype(jnp.bfloat16), _NB // _R, axis=0)
    lrep = pltpu.repeat(lo.astype(jnp.bfloat16), _NB // _R, axis=0)
    pat = (lax.broadcasted_iota(jnp.int32, (_NB, 1), 0) >> 4).astype(jnp.bfloat16)
    one = jnp.bfloat16(1.0)
    zero = jnp.bfloat16(0.0)
    ah = jnp.where(hrep == pat, one, zero)
    al = jnp.where(lrep == pat, one, zero)
    partial = lax.dot_general(ah, al, (((1,), (1,)), ((), ())),
                              preferred_element_type=jnp.float32)

    @pl.when(c == 0)
    def _():
        acc_ref[0] = partial

    @pl.when(c != 0)
    def _():
        acc_ref[0] = acc_ref[0] + partial


def _otsu_mask_kernel(gray_ref, acc_ref, out_ref):
    res = acc_ref[0]
    ia = lax.broadcasted_iota(jnp.int32, (_NB, _NB), 0)
    ib = lax.broadcasted_iota(jnp.int32, (_NB, _NB), 1)
    diag = jnp.where((ia & 15) == (ib & 15), 1.0, 0.0)
    masked = res * diag
    r16 = lax.broadcasted_iota(jnp.int32, (16, _NB), 0)
    c256 = lax.broadcasted_iota(jnp.int32, (16, _NB), 1)
    btl = jnp.where(r16 == (c256 >> 4), 1.0, 0.0)
    rr = lax.broadcasted_iota(jnp.int32, (_NB, 16), 0)
    cc = lax.broadcasted_iota(jnp.int32, (_NB, 16), 1)
    br = jnp.where((rr >> 4) == cc, 1.0, 0.0)
    t1 = lax.dot_general(btl, masked, (((1,), (0,)), ((), ())),
                         precision=_HP, preferred_element_type=jnp.float32)
    h2 = lax.dot_general(t1, br, (((1,), (0,)), ((), ())),
                         precision=_HP, preferred_element_type=jnp.float32)

    rf = lax.broadcasted_iota(jnp.int32, (16, 16), 0).astype(jnp.float32)
    cf = lax.broadcasted_iota(jnp.int32, (16, 16), 1).astype(jnp.float32)
    vmat = rf * 16.0 + cf
    tinc = jnp.where(rf <= cf, 1.0, 0.0)
    sst = jnp.where(cf < rf, 1.0, 0.0)
    ntot = jnp.sum(h2)
    p = h2 / ntot
    rowcum = lax.dot_general(p, tinc, (((1,), (0,)), ((), ())),
                             precision=_HP, preferred_element_type=jnp.float32)
    prev = lax.dot_general(sst, rowcum, (((1,), (0,)), ((), ())),
                           precision=_HP, preferred_element_type=jnp.float32)
    omega = rowcum + prev[:, 15:16]
    wgt = p * vmat
    rowcumw = lax.dot_general(wgt, tinc, (((1,), (0,)), ((), ())),
                              precision=_HP, preferred_element_type=jnp.float32)
    prevw = lax.dot_general(sst, rowcumw, (((1,), (0,)), ((), ())),
                            precision=_HP, preferred_element_type=jnp.float32)
    mu = rowcumw + prevw[:, 15:16]
    mu_t = mu[15:16, 15:16]
    denom = omega * (1.0 - omega)
    num = mu_t * omega - mu
    sigma = jnp.where(denom > 1e-12, num * num / jnp.maximum(denom, 1e-12), -1.0)
    mx = jnp.max(sigma)
    tval = jnp.min(jnp.where(sigma == mx, vmat, 3.0e5))

    gf = gray_ref[0].astype(jnp.float32)
    out_ref[0] = jnp.where(gf > tval, 1, 0).astype(jnp.int32)


def kernel(x):
    b, c, h, w = x.shape
    npix = h * w
    cols = npix // _R
    nc = cols // _CB
    x4 = x.reshape(b, c, _R, cols)
    gray, acc = pl.pallas_call(
        _hist_kernel,
        grid=(b, nc),
        in_specs=[pl.BlockSpec((1, c, _R, _CB), lambda i, j: (i, 0, 0, j))],
        out_specs=[
            pl.BlockSpec((1, _R, _CB), lambda i, j: (i, 0, j)),
            pl.BlockSpec((1, _NB, _NB), lambda i, j: (i, 0, 0)),
        ],
        out_shape=[
            jax.ShapeDtypeStruct((b, _R, cols), jnp.bfloat16),
            jax.ShapeDtypeStruct((b, _NB, _NB), jnp.float32),
        ],
        compiler_params=pltpu.CompilerParams(
            dimension_semantics=("parallel", "arbitrary")),
    )(x4)
    mask = pl.pallas_call(
        _otsu_mask_kernel,
        grid=(b,),
        in_specs=[
            pl.BlockSpec((1, _R, cols), lambda i: (i, 0, 0)),
            pl.BlockSpec((1, _NB, _NB), lambda i: (i, 0, 0)),
        ],
        out_specs=pl.BlockSpec((1, _R, cols), lambda i: (i, 0, 0)),
        out_shape=jax.ShapeDtypeStruct((b, _R, cols), jnp.int32),
        compiler_params=pltpu.CompilerParams(dimension_semantics=("parallel",)),
    )(gray, acc)
    return mask.reshape(b, h, w).astype(jnp.int64)


# speedup vs baseline: 52.5826x; 1.8046x over previous
import jax
import jax.numpy as jnp
from jax import lax
from jax.experimental import pallas as pl
from jax.experimental.pallas import tpu as pltpu

_NB = 256
_GR = 16
_BR = 128
_HP = lax.Precision.HIGHEST


def _hist_kernel(x_ref, gray_ref, hist_ref, acc_ref):
    c = pl.program_id(1)
    nc = pl.num_programs(1)
    xs = x_ref[0]
    pat = (lax.broadcasted_iota(jnp.int32, (_NB, 1), 0) >> 4).astype(jnp.bfloat16)
    one = jnp.bfloat16(1.0)
    zero = jnp.bfloat16(0.0)
    ahs = []
    als = []
    for k in range(_BR // _GR):
        sub = xs[:, k * _GR:(k + 1) * _GR, :]
        u8r = jnp.clip(jnp.floor(sub[0] * 255.0), 0.0, 255.0)
        u8g = jnp.clip(jnp.floor(sub[1] * 255.0), 0.0, 255.0)
        u8b = jnp.clip(jnp.floor(sub[2] * 255.0), 0.0, 255.0)
        grayf = jnp.clip(jnp.round(0.299 * u8r + 0.587 * u8g + 0.114 * u8b),
                         0.0, 255.0)
        gray_ref[0, k * _GR:(k + 1) * _GR, :] = grayf.astype(jnp.bfloat16)
        hi = jnp.floor(grayf * 0.0625)
        lo = grayf - hi * 16.0
        hrep = pltpu.repeat(hi.astype(jnp.bfloat16), _NB // _GR, axis=0)
        lrep = pltpu.repeat(lo.astype(jnp.bfloat16), _NB // _GR, axis=0)
        ahs.append(jnp.where(hrep == pat, one, zero))
        als.append(jnp.where(lrep == pat, one, zero))
    ah = jnp.concatenate(ahs, axis=1)
    al = jnp.concatenate(als, axis=1)
    partial = lax.dot_general(ah, al, (((1,), (1,)), ((), ())),
                              preferred_element_type=jnp.float32)

    @pl.when(c == 0)
    def _():
        acc_ref[...] = partial

    @pl.when(c != 0)
    def _():
        acc_ref[...] = acc_ref[...] + partial

    @pl.when(c == nc - 1)
    def _():
        res = acc_ref[...]
        ia = lax.broadcasted_iota(jnp.int32, (_NB, _NB), 0)
        ib = lax.broadcasted_iota(jnp.int32, (_NB, _NB), 1)
        diag = jnp.where((ia & 15) == (ib & 15), 1.0, 0.0)
        masked = res * diag
        r16 = lax.broadcasted_iota(jnp.int32, (16, _NB), 0)
        c256 = lax.broadcasted_iota(jnp.int32, (16, _NB), 1)
        btl = jnp.where(r16 == (c256 >> 4), 1.0, 0.0)
        rr = lax.broadcasted_iota(jnp.int32, (_NB, 16), 0)
        cc = lax.broadcasted_iota(jnp.int32, (_NB, 16), 1)
        br = jnp.where((rr >> 4) == cc, 1.0, 0.0)
        t1 = lax.dot_general(btl, masked, (((1,), (0,)), ((), ())),
                             precision=_HP, preferred_element_type=jnp.float32)
        h2 = lax.dot_general(t1, br, (((1,), (0,)), ((), ())),
                             precision=_HP, preferred_element_type=jnp.float32)
        hist_ref[0] = h2


def _otsu_mask_kernel(gray_ref, hist_ref, out_ref):
    h2 = hist_ref[0]
    rf = lax.broadcasted_iota(jnp.int32, (16, 16), 0).astype(jnp.float32)
    cf = lax.broadcasted_iota(jnp.int32, (16, 16), 1).astype(jnp.float32)
    vmat = rf * 16.0 + cf
    tinc = jnp.where(rf <= cf, 1.0, 0.0)
    sst = jnp.where(cf < rf, 1.0, 0.0)
    ntot = jnp.sum(h2)
    p = h2 / ntot
    rowcum = lax.dot_general(p, tinc, (((1,), (0,)), ((), ())),
                             precision=_HP, preferred_element_type=jnp.float32)
    prev = lax.dot_general(sst, rowcum, (((1,), (0,)), ((), ())),
                           precision=_HP, preferred_element_type=jnp.float32)
    omega = rowcum + prev[:, 15:16]
    wgt = p * vmat
    rowcumw = lax.dot_general(wgt, tinc, (((1,), (0,)), ((), ())),
                              precision=_HP, preferred_element_type=jnp.float32)
    prevw = lax.dot_general(sst, rowcumw, (((1,), (0,)), ((), ())),
                            precision=_HP, preferred_element_type=jnp.float32)
    mu = rowcumw + prevw[:, 15:16]
    mu_t = mu[15:16, 15:16]
    denom = omega * (1.0 - omega)
    num = mu_t * omega - mu
    sigma = jnp.where(denom > 1e-12, num * num / jnp.maximum(denom, 1e-12), -1.0)
    mx = jnp.max(sigma)
    tval = jnp.min(jnp.where(sigma == mx, vmat, 3.0e5))

    gf = gray_ref[0].astype(jnp.float32)
    out_ref[0] = jnp.where(gf > tval, 1, 0).astype(jnp.int32)


def kernel(x):
    b, c, h, w = x.shape
    nc = h // _BR
    gray, hist = pl.pallas_call(
        _hist_kernel,
        grid=(b, nc),
        in_specs=[pl.BlockSpec((1, c, _BR, w), lambda i, j: (i, 0, j, 0))],
        out_specs=[
            pl.BlockSpec((1, _BR, w), lambda i, j: (i, j, 0)),
            pl.BlockSpec((1, 16, 16), lambda i, j: (i, 0, 0)),
        ],
        out_shape=[
            jax.ShapeDtypeStruct((b, h, w), jnp.bfloat16),
            jax.ShapeDtypeStruct((b, 16, 16), jnp.float32),
        ],
        scratch_shapes=[pltpu.VMEM((_NB, _NB), jnp.float32)],
        compiler_params=pltpu.CompilerParams(
            dimension_semantics=("parallel", "arbitrary")),
    )(x)
    mask = pl.pallas_call(
        _otsu_mask_kernel,
        grid=(b,),
        in_specs=[
            pl.BlockSpec((1, h, w), lambda i: (i, 0, 0)),
            pl.BlockSpec((1, 16, 16), lambda i: (i, 0, 0)),
        ],
        out_specs=pl.BlockSpec((1, h, w), lambda i: (i, 0, 0)),
        out_shape=jax.ShapeDtypeStruct((b, h, w), jnp.int32),
        compiler_params=pltpu.CompilerParams(dimension_semantics=("parallel",)),
    )(gray, hist)
    return mask.astype(jnp.int64)
